# baseline (speedup 1.0000x reference)
import functools

import jax
import jax.numpy as jnp
from jax.experimental import pallas as pl
from jax.experimental.pallas import tpu as pltpu

_BM = 224


def _gcn_block_kernel(x_ref, w_ref, top_ref, bot_ref, b_ref, out_ref,
                      support_ref):
    @pl.when(pl.program_id(0) == 0)
    def _compute_support():
        support_ref[...] = jnp.dot(
            x_ref[...], w_ref[...], preferred_element_type=jnp.float32
        )

    support = support_ref[...]
    bias = b_ref[...]
    acc_t = jnp.dot(top_ref[0], support, preferred_element_type=jnp.float32)
    out_ref[0] = jnp.maximum(acc_t + bias, 0.0)
    acc_b = jnp.dot(bot_ref[0], support, preferred_element_type=jnp.float32)
    out_ref[1] = jnp.maximum(acc_b + bias, 0.0)


@functools.partial(jax.jit, static_argnames=())
def kernel(x, adj, weight, bias):
    n, d_in = x.shape
    d_out = weight.shape[1]
    nh = n // 2
    adj3 = adj.reshape(2, nh, n)
    bias2d = bias.reshape(1, d_out)
    grid = (pl.cdiv(nh, _BM),)
    out = pl.pallas_call(
        _gcn_block_kernel,
        grid=grid,
        in_specs=[
            pl.BlockSpec((n, d_in), lambda i: (0, 0)),
            pl.BlockSpec((d_in, d_out), lambda i: (0, 0)),
            pl.BlockSpec((1, _BM, n), lambda i: (0, i, 0)),
            pl.BlockSpec((1, _BM, n), lambda i: (1, i, 0)),
            pl.BlockSpec((1, d_out), lambda i: (0, 0)),
        ],
        out_specs=pl.BlockSpec((2, _BM, d_out), lambda i: (0, i, 0)),
        out_shape=jax.ShapeDtypeStruct((2, nh, d_out), jnp.float32),
        scratch_shapes=[pltpu.VMEM((n, d_in), jnp.float32)],
    )(x, weight, adj3, adj3, bias2d)
    return out.reshape(n, d_out)


# speedup vs baseline: 1.0185x; 1.0185x over previous
import functools

import jax
import jax.numpy as jnp
from jax.experimental import pallas as pl
from jax.experimental.pallas import tpu as pltpu

_N = 10000
_BM = 232


def _gcn_block_kernel(x_ref, w_ref, adj_ref, b_ref, out_ref, support_ref):
    @pl.when(pl.program_id(0) == 0)
    def _compute_support():
        support_ref[...] = jnp.dot(
            x_ref[...], w_ref[...], preferred_element_type=jnp.float32
        )

    acc = jnp.dot(adj_ref[...], support_ref[...],
                  preferred_element_type=jnp.float32)
    out_ref[...] = jnp.maximum(acc + b_ref[...], 0.0)


@functools.partial(jax.jit, static_argnames=())
def kernel(x, adj, weight, bias):
    n, d_in = x.shape
    d_out = weight.shape[1]
    bias2d = bias.reshape(1, d_out)
    grid = (pl.cdiv(n, _BM),)
    out = pl.pallas_call(
        _gcn_block_kernel,
        grid=grid,
        in_specs=[
            pl.BlockSpec((n, d_in), lambda i: (0, 0)),
            pl.BlockSpec((d_in, d_out), lambda i: (0, 0)),
            pl.BlockSpec((_BM, n), lambda i: (i, 0)),
            pl.BlockSpec((1, d_out), lambda i: (0, 0)),
        ],
        out_specs=pl.BlockSpec((_BM, d_out), lambda i: (i, 0)),
        out_shape=jax.ShapeDtypeStruct((n, d_out), jnp.float32),
        scratch_shapes=[pltpu.VMEM((n, d_in), jnp.float32)],
    )(x, weight, adj, bias2d)
    return out


# speedup vs baseline: 1.0290x; 1.0103x over previous
import functools

import jax
import jax.numpy as jnp
from jax.experimental import pallas as pl
from jax.experimental.pallas import tpu as pltpu

_N = 10000
_BM = 224


def _gcn_block_kernel(x_ref, w_ref, adj_ref, b_ref, out_ref, support_ref):
    @pl.when(pl.program_id(0) == 0)
    def _compute_support():
        support_ref[...] = jnp.dot(
            x_ref[...], w_ref[...], preferred_element_type=jnp.float32
        )

    acc = jnp.dot(adj_ref[...], support_ref[...],
                  preferred_element_type=jnp.float32)
    out_ref[...] = jnp.maximum(acc + b_ref[...], 0.0)


@functools.partial(jax.jit, static_argnames=())
def kernel(x, adj, weight, bias):
    n, d_in = x.shape
    d_out = weight.shape[1]
    bias2d = bias.reshape(1, d_out)
    grid = (pl.cdiv(n, _BM),)
    out = pl.pallas_call(
        _gcn_block_kernel,
        grid=grid,
        in_specs=[
            pl.BlockSpec((n, d_in), lambda i: (0, 0)),
            pl.BlockSpec((d_in, d_out), lambda i: (0, 0)),
            pl.BlockSpec((_BM, n), lambda i: (i, 0)),
            pl.BlockSpec((1, d_out), lambda i: (0, 0)),
        ],
        out_specs=pl.BlockSpec((_BM, d_out), lambda i: (i, 0)),
        out_shape=jax.ShapeDtypeStruct((n, d_out), jnp.float32),
        scratch_shapes=[pltpu.VMEM((n, d_in), jnp.float32)],
    )(x, weight, adj, bias2d)
    return out
